# baseline (speedup 1.0000x reference)
import functools

import jax
import jax.numpy as jnp
from jax import lax
from jax.experimental import pallas as pl
from jax.experimental.pallas import tpu as pltpu
from jax.experimental.pallas import tpu_sc as plsc

B = 16384
E = 32
H = 128
NM = 100000

NC = 2
NS = 16
NW = NC * NS
BPW = B // NW
L = 16
GS = 8
NG = BPW // GS


@functools.cache
def _make_user_gather():
    mesh = plsc.VectorSubcoreMesh(core_axis_name="c", subcore_axis_name="s")

    @functools.partial(
        pl.kernel,
        mesh=mesh,
        out_type=jax.ShapeDtypeStruct((B, E), jnp.float32),
        scratch_types=[
            pltpu.VMEM((BPW,), jnp.int32),
            pltpu.VMEM((NG * L,), jnp.int32),
            pltpu.VMEM((GS, E, H), jnp.float32),
            pltpu.VMEM((GS, E, H), jnp.float32),
            pltpu.VMEM((GS, E), jnp.float32),
            pltpu.VMEM((GS, E), jnp.float32),
            pltpu.SemaphoreType.DMA,
            pltpu.SemaphoreType.DMA,
            pltpu.SemaphoreType.DMA,
        ],
        compiler_params=pltpu.CompilerParams(needs_layout_passes=False),
    )
    def _gather(users, utabT, out, idx_v, idx2, bufA, bufB,
                stgA, stgB, semA, semB, wsem):
        wid = lax.axis_index("s") * NC + lax.axis_index("c")
        base = wid * BPW
        lane = lax.iota(jnp.int32, L)

        pltpu.sync_copy(users.at[pl.ds(base, BPW)], idx_v)

        dstpos = (lax.shift_right_logical(lane, 3) * L) + (lane & (GS - 1))

        def _spread(q, carry):
            vecs = idx_v[pl.ds(q * L, L)]
            plsc.store_scatter(idx2, [q * ((L // GS) * L) + dstpos], vecs)
            return carry

        lax.fori_loop(0, BPW // L, _spread, 0)

        def fire(g, buf, sem):
            vec = idx2[pl.ds(g * L, L)]
            for k in range(GS):
                i = vec[k]
                c128 = pl.multiple_of(i & ~(H - 1), H)
                pltpu.async_copy(utabT.at[:, pl.ds(c128, H)], buf.at[k], sem)

        def handle(g, buf, sem, stg):
            for k in range(GS):
                pltpu.make_async_copy(utabT.at[:, pl.ds(0, H)],
                                      buf.at[k], sem).wait()
            vec = idx2[pl.ds(g * L, L)]
            for k in range(GS):
                i = vec[k]
                remv = jnp.full((L,), i & (H - 1), jnp.int32)
                kv = jnp.full((L,), k, jnp.int32)
                v0 = plsc.load_gather(buf, [kv, lane, remv])
                v1 = plsc.load_gather(buf, [kv, lane + L, remv])
                stg[k, pl.ds(0, L)] = v0
                stg[k, pl.ds(L, L)] = v1
            pltpu.async_copy(stg, out.at[pl.ds(base + g * GS, GS)], wsem)

        lanes = ((bufA, semA, stgA), (bufB, semB, stgB))
        NBUF = len(lanes)

        def _body(p, carry):
            for j, (buf, sem, stg) in enumerate(lanes):
                g = NBUF * p + j

                @pl.when(g >= NBUF)
                def _(stg=stg):
                    pltpu.make_async_copy(out.at[pl.ds(0, GS)], stg,
                                          wsem).wait()
                handle(g, buf, sem, stg)

                @pl.when(g + NBUF < NG)
                def _(g=g, buf=buf, sem=sem):
                    fire(g + NBUF, buf, sem)
            return carry

        for j, (buf, sem, _) in enumerate(lanes):
            fire(j, buf, sem)

        lax.fori_loop(0, NG // NBUF, _body, 0)
        for _, _, stg in lanes:
            pltpu.make_async_copy(out.at[pl.ds(0, GS)], stg, wsem).wait()

    return _gather


HB = B // 2
MPW = HB // NW
MCH = 128


@functools.cache
def _make_pm_gather():
    mesh = plsc.VectorSubcoreMesh(core_axis_name="c", subcore_axis_name="s")

    @functools.partial(
        pl.kernel,
        mesh=mesh,
        out_type=jax.ShapeDtypeStruct((HB, H), jnp.float32),
        scratch_types=[
            pltpu.VMEM((MPW,), jnp.int32),
            pltpu.VMEM((MPW, H), jnp.float32),
            pltpu.SemaphoreType.DMA,
        ],
        compiler_params=pltpu.CompilerParams(needs_layout_passes=False),
    )
    def _gather(movies_half, pm_tab, out, idx_v, rows_v, sem):
        wid = lax.axis_index("s") * NC + lax.axis_index("c")
        base = wid * MPW
        pltpu.sync_copy(movies_half.at[pl.ds(base, MPW)], idx_v)
        copies = [
            pltpu.async_copy(
                pm_tab.at[idx_v.at[pl.ds(j * MCH, MCH)]],
                rows_v.at[pl.ds(j * MCH, MCH)], sem)
            for j in range(MPW // MCH)
        ]
        for c in copies:
            c.wait()
        pltpu.sync_copy(rows_v, out.at[pl.ds(base, MPW)])

    return _gather


NMP = 102400
BLKM = 12800


def _proj_body(mT, w1m, o):
    o[...] = jax.lax.dot_general(
        mT[...], w1m[...], (((0,), (0,)), ((), ())),
        preferred_element_type=jnp.float32)


def _movie_proj(mtabT, w1m):
    return pl.pallas_call(
        _proj_body,
        grid=(NMP // BLKM,),
        in_specs=[
            pl.BlockSpec((E, BLKM), lambda i: (0, i)),
            pl.BlockSpec((E, H), lambda i: (0, 0)),
        ],
        out_specs=pl.BlockSpec((BLKM, H), lambda i: (i, 0)),
        out_shape=jax.ShapeDtypeStruct((NMP, H), jnp.float32),
    )(mtabT, w1m)


BLK = 2048


def _mlp_body(u, pm, w1u, b1, w2, b2, o):
    h = jnp.dot(u[...], w1u[...], preferred_element_type=jnp.float32)
    h = jnp.maximum(h + pm[...] + b1[...], 0.0)
    o[...] = jnp.dot(h, w2[...], preferred_element_type=jnp.float32) + b2[...]


def _mlp(u, pm, w1u, b1, w2, b2):
    n = u.shape[0]
    return pl.pallas_call(
        _mlp_body,
        grid=(n // BLK,),
        in_specs=[
            pl.BlockSpec((BLK, E), lambda i: (i, 0)),
            pl.BlockSpec((BLK, H), lambda i: (i, 0)),
            pl.BlockSpec((E, H), lambda i: (0, 0)),
            pl.BlockSpec((1, H), lambda i: (0, 0)),
            pl.BlockSpec((H, 1), lambda i: (0, 0)),
            pl.BlockSpec((1, 1), lambda i: (0, 0)),
        ],
        out_specs=pl.BlockSpec((BLK, 1), lambda i: (i, 0)),
        out_shape=jax.ShapeDtypeStruct((n, 1), jnp.float32),
    )(u, pm, w1u, b1, w2, b2)


def kernel(users, movies, user_table, movie_table, W1, b1, W2, b2):
    u_emb = _make_user_gather()(users, user_table.T)
    mtabT = jnp.pad(movie_table.T, ((0, 0), (0, NMP - NM)))
    pm_tab = _movie_proj(mtabT, W1[E:])
    dep = (u_emb[0, 0] * 0.0).astype(jnp.int32)
    movies = movies + dep
    pm1 = _make_pm_gather()(movies[:HB], pm_tab)
    pm2 = _make_pm_gather()(movies[HB:], pm_tab)
    w1u = W1[:E]
    b1r = b1.reshape(1, H)
    b2r = b2.reshape(1, 1)
    o1 = _mlp(u_emb[:HB], pm1, w1u, b1r, W2, b2r)
    o2 = _mlp(u_emb[HB:], pm2, w1u, b1r, W2, b2r)
    return jnp.concatenate([o1, o2], axis=0)


# speedup vs baseline: 1.0487x; 1.0487x over previous
import functools

import jax
import jax.numpy as jnp
from jax import lax
from jax.experimental import pallas as pl
from jax.experimental.pallas import tpu as pltpu
from jax.experimental.pallas import tpu_sc as plsc

B = 16384
E = 32
H = 128
NM = 100000

NC = 2
NS = 16
NW = NC * NS
BPW = B // NW
L = 16
GS = 4
NG = BPW // GS


@functools.cache
def _make_user_gather():
    mesh = plsc.VectorSubcoreMesh(core_axis_name="c", subcore_axis_name="s")

    @functools.partial(
        pl.kernel,
        mesh=mesh,
        out_type=jax.ShapeDtypeStruct((B, E), jnp.float32),
        scratch_types=[
            pltpu.VMEM((BPW,), jnp.int32),
            pltpu.VMEM((NG * L,), jnp.int32),
            pltpu.VMEM((GS, E, H), jnp.float32),
            pltpu.VMEM((GS, E, H), jnp.float32),
            pltpu.VMEM((GS, E, H), jnp.float32),
            pltpu.VMEM((GS, E, H), jnp.float32),
            pltpu.VMEM((GS, E), jnp.float32),
            pltpu.VMEM((GS, E), jnp.float32),
            pltpu.VMEM((GS, E), jnp.float32),
            pltpu.VMEM((GS, E), jnp.float32),
            pltpu.SemaphoreType.DMA,
            pltpu.SemaphoreType.DMA,
            pltpu.SemaphoreType.DMA,
            pltpu.SemaphoreType.DMA,
            pltpu.SemaphoreType.DMA,
        ],
        compiler_params=pltpu.CompilerParams(needs_layout_passes=False),
    )
    def _gather(users, utabT, out, idx_v, idx2, bufA, bufB, bufC, bufD,
                stgA, stgB, stgC, stgD, semA, semB, semC, semD, wsem):
        wid = lax.axis_index("s") * NC + lax.axis_index("c")
        base = wid * BPW
        lane = lax.iota(jnp.int32, L)

        pltpu.sync_copy(users.at[pl.ds(base, BPW)], idx_v)

        dstpos = (lax.shift_right_logical(lane, 2) * L) + (lane & (GS - 1))

        def _spread(q, carry):
            vecs = idx_v[pl.ds(q * L, L)]
            plsc.store_scatter(idx2, [q * (4 * L) + dstpos], vecs)
            return carry

        lax.fori_loop(0, BPW // L, _spread, 0)

        def fire(g, buf, sem):
            vec = idx2[pl.ds(g * L, L)]
            for k in range(GS):
                i = vec[k]
                c128 = pl.multiple_of(i & ~(H - 1), H)
                pltpu.async_copy(utabT.at[:, pl.ds(c128, H)], buf.at[k], sem)

        def handle(g, buf, sem, stg):
            for k in range(GS):
                pltpu.make_async_copy(utabT.at[:, pl.ds(0, H)],
                                      buf.at[k], sem).wait()
            vec = idx2[pl.ds(g * L, L)]
            for k in range(GS):
                i = vec[k]
                remv = jnp.full((L,), i & (H - 1), jnp.int32)
                kv = jnp.full((L,), k, jnp.int32)
                v0 = plsc.load_gather(buf, [kv, lane, remv])
                v1 = plsc.load_gather(buf, [kv, lane + L, remv])
                stg[k, pl.ds(0, L)] = v0
                stg[k, pl.ds(L, L)] = v1
            pltpu.async_copy(stg, out.at[pl.ds(base + g * GS, GS)], wsem)

        lanes = ((bufA, semA, stgA), (bufB, semB, stgB),
                 (bufC, semC, stgC), (bufD, semD, stgD))
        NBUF = len(lanes)

        def _body(p, carry):
            for j, (buf, sem, stg) in enumerate(lanes):
                g = NBUF * p + j

                @pl.when(g >= NBUF)
                def _(stg=stg):
                    pltpu.make_async_copy(out.at[pl.ds(0, GS)], stg,
                                          wsem).wait()
                handle(g, buf, sem, stg)

                @pl.when(g + NBUF < NG)
                def _(g=g, buf=buf, sem=sem):
                    fire(g + NBUF, buf, sem)
            return carry

        for j, (buf, sem, _) in enumerate(lanes):
            fire(j, buf, sem)

        lax.fori_loop(0, NG // NBUF, _body, 0)
        for _, _, stg in lanes:
            pltpu.make_async_copy(out.at[pl.ds(0, GS)], stg, wsem).wait()

    return _gather


HB = B // 2
MPW = HB // NW
MCH = 128


@functools.cache
def _make_pm_gather():
    mesh = plsc.VectorSubcoreMesh(core_axis_name="c", subcore_axis_name="s")

    @functools.partial(
        pl.kernel,
        mesh=mesh,
        out_type=jax.ShapeDtypeStruct((HB, H), jnp.float32),
        scratch_types=[
            pltpu.VMEM((MPW,), jnp.int32),
            pltpu.VMEM((MPW, H), jnp.float32),
            pltpu.SemaphoreType.DMA,
        ],
        compiler_params=pltpu.CompilerParams(needs_layout_passes=False),
    )
    def _gather(movies_half, pm_tab, out, idx_v, rows_v, sem):
        wid = lax.axis_index("s") * NC + lax.axis_index("c")
        base = wid * MPW
        pltpu.sync_copy(movies_half.at[pl.ds(base, MPW)], idx_v)
        copies = [
            pltpu.async_copy(
                pm_tab.at[idx_v.at[pl.ds(j * MCH, MCH)]],
                rows_v.at[pl.ds(j * MCH, MCH)], sem)
            for j in range(MPW // MCH)
        ]
        for c in copies:
            c.wait()
        pltpu.sync_copy(rows_v, out.at[pl.ds(base, MPW)])

    return _gather


NMP = 102400
BLKM = 12800


def _proj_body(mT, w1m, o):
    o[...] = jax.lax.dot_general(
        mT[...], w1m[...], (((0,), (0,)), ((), ())),
        preferred_element_type=jnp.float32)


def _movie_proj(mtabT, w1m):
    return pl.pallas_call(
        _proj_body,
        grid=(NMP // BLKM,),
        in_specs=[
            pl.BlockSpec((E, BLKM), lambda i: (0, i)),
            pl.BlockSpec((E, H), lambda i: (0, 0)),
        ],
        out_specs=pl.BlockSpec((BLKM, H), lambda i: (i, 0)),
        out_shape=jax.ShapeDtypeStruct((NMP, H), jnp.float32),
    )(mtabT, w1m)


BLK = 2048


def _mlp_body(u, pm, w1u, b1, w2, b2, o):
    h = jnp.dot(u[...], w1u[...], preferred_element_type=jnp.float32)
    h = jnp.maximum(h + pm[...] + b1[...], 0.0)
    o[...] = jnp.dot(h, w2[...], preferred_element_type=jnp.float32) + b2[...]


def _mlp(u, pm, w1u, b1, w2, b2):
    n = u.shape[0]
    return pl.pallas_call(
        _mlp_body,
        grid=(n // BLK,),
        in_specs=[
            pl.BlockSpec((BLK, E), lambda i: (i, 0)),
            pl.BlockSpec((BLK, H), lambda i: (i, 0)),
            pl.BlockSpec((E, H), lambda i: (0, 0)),
            pl.BlockSpec((1, H), lambda i: (0, 0)),
            pl.BlockSpec((H, 1), lambda i: (0, 0)),
            pl.BlockSpec((1, 1), lambda i: (0, 0)),
        ],
        out_specs=pl.BlockSpec((BLK, 1), lambda i: (i, 0)),
        out_shape=jax.ShapeDtypeStruct((n, 1), jnp.float32),
    )(u, pm, w1u, b1, w2, b2)


def kernel(users, movies, user_table, movie_table, W1, b1, W2, b2):
    u_emb = _make_user_gather()(users, user_table.T)
    mtabT = jnp.pad(movie_table.T, ((0, 0), (0, NMP - NM)))
    pm_tab = _movie_proj(mtabT, W1[E:])
    dep = (u_emb[0, 0] * 0.0).astype(jnp.int32)
    movies = movies + dep
    pm1 = _make_pm_gather()(movies[:HB], pm_tab)
    pm2 = _make_pm_gather()(movies[HB:], pm_tab)
    w1u = W1[:E]
    b1r = b1.reshape(1, H)
    b2r = b2.reshape(1, 1)
    o1 = _mlp(u_emb[:HB], pm1, w1u, b1r, W2, b2r)
    o2 = _mlp(u_emb[HB:], pm2, w1u, b1r, W2, b2r)
    return jnp.concatenate([o1, o2], axis=0)
